# baseline (speedup 1.0000x reference)
import functools

import jax
import jax.numpy as jnp
from jax import lax
from jax.experimental import pallas as pl
from jax.experimental.pallas import tpu as pltpu
from jax.experimental.pallas import tpu_sc as plsc

N = 10000
D = 128
NC = 2
NS = 16
NW = NC * NS
L = 16
K = 128
CH = 80
NPAIR = CH // 2
EPW = CH * K
EP = EPW * NW
TOTCH = NW * CH
ZR = 640
R16 = ZR * NS
NDQ = NPAIR // 2
BR = 1000
GRID = N // BR

_mesh = plsc.VectorSubcoreMesh(
    core_axis_name="c", subcore_axis_name="s", num_cores=NC, num_subcores=NS
)


@functools.partial(
    pl.kernel,
    out_type=jax.ShapeDtypeStruct((NC, R16, D), jnp.float32),
    mesh=_mesh,
    scratch_types=[
        pltpu.VMEM((2, K), jnp.int32),
        pltpu.VMEM((2, K), jnp.int32),
        pltpu.VMEM((K, D), jnp.float32),
        pltpu.VMEM_SHARED((R16, D), jnp.float32),
        pltpu.SemaphoreType.DMA,
        pltpu.SemaphoreType.DMA,
    ],
)
def _deg_kernel(rc_hbm, deg_hbm, idx0, idx1, msg_v, deg_sh, sem0, sem1):
    cid = lax.axis_index("c")
    sid = lax.axis_index("s")
    wid = sid * NC + cid
    cb = wid * CH

    def fill(val):
        def body(i, carry):
            for j in range(D // L):
                msg_v[i, pl.ds(j * L, L)] = jnp.full((L,), val, jnp.float32)
            return carry

        lax.fori_loop(0, K, body, 0)

    fill(0.0)
    for t in range(ZR // K):
        pltpu.sync_copy(msg_v, deg_sh.at[pl.ds(sid * ZR + t * K, K)])
    fill(1.0)
    plsc.subcore_barrier()

    pltpu.sync_copy(rc_hbm.at[cb], idx0)

    def body(i, carry):
        c0 = cb + 2 * i
        pltpu.async_copy(rc_hbm.at[c0 + 1], idx1, sem1)
        pltpu.sync_copy(msg_v, deg_sh.at[idx0.at[1]], add=True)

        @pl.when(i < NPAIR - 1)
        def _():
            pltpu.async_copy(rc_hbm.at[c0 + 2], idx0, sem0)

        pltpu.make_async_copy(rc_hbm.at[c0 + 1], idx1, sem1).wait()
        pltpu.sync_copy(msg_v, deg_sh.at[idx1.at[1]], add=True)

        @pl.when(i < NPAIR - 1)
        def _():
            pltpu.make_async_copy(rc_hbm.at[c0 + 2], idx0, sem0).wait()

        return carry

    lax.fori_loop(0, NPAIR, body, 0)
    plsc.subcore_barrier()

    pltpu.sync_copy(
        deg_sh.at[pl.ds(sid * ZR, ZR)], deg_hbm.at[cid].at[pl.ds(sid * ZR, ZR)]
    )


@functools.partial(
    pl.kernel,
    out_type=jax.ShapeDtypeStruct((NC, R16, D), jnp.float32),
    mesh=_mesh,
    scratch_types=[
        pltpu.VMEM((2, 2, K), jnp.int32),
        pltpu.VMEM((2, 2, K), jnp.int32),
        pltpu.VMEM((K, D), jnp.float32),
        pltpu.VMEM((K, D), jnp.float32),
        pltpu.VMEM_SHARED((R16, D), jnp.float32),
        pltpu.SemaphoreType.DMA,
        pltpu.SemaphoreType.DMA,
        pltpu.SemaphoreType.DMA,
        pltpu.SemaphoreType.DMA,
        pltpu.SemaphoreType.DMA,
        pltpu.SemaphoreType.DMA,
    ],
)
def _scatter_kernel(
    y_hbm, rcp_hbm, acc_hbm, pA, pB, msg0, msg1, acc_sh,
    sem0, sem1, semA, semB, sems0, sems1,
):
    cid = lax.axis_index("c")
    sid = lax.axis_index("s")
    wid = sid * NC + cid
    pb = wid * NPAIR

    def fill_zeros(i, carry):
        for j in range(D // L):
            msg0[i, pl.ds(j * L, L)] = jnp.zeros((L,), jnp.float32)
        return carry

    lax.fori_loop(0, K, fill_zeros, 0)
    for t in range(ZR // K):
        pltpu.sync_copy(msg0, acc_sh.at[pl.ds(sid * ZR + t * K, K)])
    plsc.subcore_barrier()

    pltpu.sync_copy(rcp_hbm.at[pb], pA)
    pltpu.async_copy(rcp_hbm.at[pb + 1], pB, semB)
    pltpu.async_copy(y_hbm.at[pA.at[0, 0]], msg0, sem0)
    pltpu.async_copy(y_hbm.at[pA.at[1, 0]], msg1, sem1)

    def dq(d, carry):
        pltpu.make_async_copy(y_hbm.at[pA.at[0, 0]], msg0, sem0).wait()
        pltpu.async_copy(msg0, acc_sh.at[pA.at[0, 1]], sems0, add=True)
        pltpu.make_async_copy(y_hbm.at[pA.at[1, 0]], msg1, sem1).wait()
        pltpu.async_copy(msg1, acc_sh.at[pA.at[1, 1]], sems1, add=True)
        pltpu.make_async_copy(rcp_hbm.at[pb], pB, semB).wait()
        pltpu.make_async_copy(msg0, acc_sh.at[pA.at[0, 1]], sems0).wait()
        pltpu.async_copy(y_hbm.at[pB.at[0, 0]], msg0, sem0)
        pltpu.make_async_copy(msg1, acc_sh.at[pA.at[1, 1]], sems1).wait()
        pltpu.async_copy(y_hbm.at[pB.at[1, 0]], msg1, sem1)

        @pl.when(d < NDQ - 1)
        def _():
            pltpu.async_copy(rcp_hbm.at[pb + 2 * d + 2], pA, semA)

        pltpu.make_async_copy(y_hbm.at[pB.at[0, 0]], msg0, sem0).wait()
        pltpu.async_copy(msg0, acc_sh.at[pB.at[0, 1]], sems0, add=True)
        pltpu.make_async_copy(y_hbm.at[pB.at[1, 0]], msg1, sem1).wait()
        pltpu.async_copy(msg1, acc_sh.at[pB.at[1, 1]], sems1, add=True)

        @pl.when(d < NDQ - 1)
        def _():
            pltpu.make_async_copy(rcp_hbm.at[pb], pA, semA).wait()
            pltpu.make_async_copy(msg0, acc_sh.at[pB.at[0, 1]], sems0).wait()
            pltpu.async_copy(y_hbm.at[pA.at[0, 0]], msg0, sem0)
            pltpu.make_async_copy(msg1, acc_sh.at[pB.at[1, 1]], sems1).wait()
            pltpu.async_copy(y_hbm.at[pA.at[1, 0]], msg1, sem1)
            pltpu.async_copy(rcp_hbm.at[pb + 2 * d + 3], pB, semB)

        return carry

    lax.fori_loop(0, NDQ, dq, 0)
    pltpu.make_async_copy(msg0, acc_sh.at[pB.at[0, 1]], sems0).wait()
    pltpu.make_async_copy(msg1, acc_sh.at[pB.at[1, 1]], sems1).wait()
    plsc.subcore_barrier()
    pltpu.sync_copy(
        acc_sh.at[pl.ds(sid * ZR, ZR)], acc_hbm.at[cid].at[pl.ds(sid * ZR, ZR)]
    )


def _dis_from(deg_ref):
    s = deg_ref[0, :, 0:1] + deg_ref[1, :, 0:1] + 1.0
    return lax.rsqrt(s)


def _prep_body(x_ref, w_ref, deg_ref, y_ref):
    dis = _dis_from(deg_ref)
    y_ref[...] = (
        jnp.dot(x_ref[...], w_ref[...], preferred_element_type=jnp.float32) * dis
    )


_prep = pl.pallas_call(
    _prep_body,
    grid=(GRID,),
    in_specs=[
        pl.BlockSpec((BR, D), lambda m: (m, 0)),
        pl.BlockSpec((D, D), lambda m: (0, 0)),
        pl.BlockSpec((NC, BR, D), lambda m: (0, m, 0)),
    ],
    out_specs=pl.BlockSpec((BR, D), lambda m: (m, 0)),
    out_shape=jax.ShapeDtypeStruct((N, D), jnp.float32),
)


def _comb_mm_body(acc_ref, y_ref, deg_ref, b_ref, w_ref, out_ref):
    dis = _dis_from(deg_ref)
    h = jnp.maximum(
        (acc_ref[0] + acc_ref[1] + y_ref[...]) * dis + b_ref[...], 0.0
    )
    out_ref[...] = (
        jnp.dot(h, w_ref[...], preferred_element_type=jnp.float32) * dis
    )


_comb_mm = pl.pallas_call(
    _comb_mm_body,
    grid=(GRID,),
    in_specs=[
        pl.BlockSpec((NC, BR, D), lambda m: (0, m, 0)),
        pl.BlockSpec((BR, D), lambda m: (m, 0)),
        pl.BlockSpec((NC, BR, D), lambda m: (0, m, 0)),
        pl.BlockSpec((1, D), lambda m: (0, 0)),
        pl.BlockSpec((D, D), lambda m: (0, 0)),
    ],
    out_specs=pl.BlockSpec((BR, D), lambda m: (m, 0)),
    out_shape=jax.ShapeDtypeStruct((N, D), jnp.float32),
)


def _comb_body(acc_ref, y_ref, deg_ref, b_ref, out_ref):
    dis = _dis_from(deg_ref)
    out_ref[...] = jnp.maximum(
        (acc_ref[0] + acc_ref[1] + y_ref[...]) * dis + b_ref[...], 0.0
    )


_comb = pl.pallas_call(
    _comb_body,
    grid=(GRID,),
    in_specs=[
        pl.BlockSpec((NC, BR, D), lambda m: (0, m, 0)),
        pl.BlockSpec((BR, D), lambda m: (m, 0)),
        pl.BlockSpec((NC, BR, D), lambda m: (0, m, 0)),
        pl.BlockSpec((1, D), lambda m: (0, 0)),
    ],
    out_specs=pl.BlockSpec((BR, D), lambda m: (m, 0)),
    out_shape=jax.ShapeDtypeStruct((N, D), jnp.float32),
)


def kernel(x, edge_index, W1, b1, W2, b2):
    row = edge_index[0]
    col = edge_index[1]
    e = row.shape[0]
    pad = EP - e
    sprd = jnp.arange(pad, dtype=jnp.int32)
    row_p = jnp.concatenate([row, sprd % N])
    col_p = jnp.concatenate([col, N + sprd % (R16 - N)])
    rc = jnp.stack([row_p.reshape(TOTCH, K), col_p.reshape(TOTCH, K)], axis=1)
    rcp = rc.reshape(TOTCH // 2, 2, 2, K)

    degpair = _deg_kernel(rc)
    b1r = b1.reshape(1, D)
    b2r = b2.reshape(1, D)

    y1 = _prep(x, W1, degpair)
    accp1 = _scatter_kernel(y1, rcp)
    y2 = _comb_mm(accp1, y1, degpair, b1r, W2)
    accp2 = _scatter_kernel(y2, rcp)
    return _comb(accp2, y2, degpair, b2r)


# speedup vs baseline: 1.2207x; 1.2207x over previous
import functools

import jax
import jax.numpy as jnp
from jax import lax
from jax.experimental import pallas as pl
from jax.experimental.pallas import tpu as pltpu
from jax.experimental.pallas import tpu_sc as plsc

N = 10000
D = 128
NC = 2
NS = 16
NW = NC * NS
L = 16
K = 128
CH = 80
NPAIR = CH // 2
EPW = CH * K
EP = EPW * NW
TOTCH = NW * CH
ZR = 640
R16 = ZR * NS
NQUAD = NPAIR // 2
BR = 1000
GRID = N // BR

_mesh = plsc.VectorSubcoreMesh(
    core_axis_name="c", subcore_axis_name="s", num_cores=NC, num_subcores=NS
)


@functools.partial(
    pl.kernel,
    out_type=jax.ShapeDtypeStruct((NC, R16, D), jnp.float32),
    mesh=_mesh,
    scratch_types=[
        pltpu.VMEM((2, K), jnp.int32),
        pltpu.VMEM((2, K), jnp.int32),
        pltpu.VMEM((K, D), jnp.float32),
        pltpu.VMEM_SHARED((R16, D), jnp.float32),
        pltpu.SemaphoreType.DMA,
        pltpu.SemaphoreType.DMA,
    ],
)
def _deg_kernel(rc_hbm, deg_hbm, idx0, idx1, msg_v, deg_sh, sem0, sem1):
    cid = lax.axis_index("c")
    sid = lax.axis_index("s")
    wid = sid * NC + cid
    cb = wid * CH

    def fill(val):
        def body(i, carry):
            for j in range(D // L):
                msg_v[i, pl.ds(j * L, L)] = jnp.full((L,), val, jnp.float32)
            return carry

        lax.fori_loop(0, K, body, 0)

    fill(0.0)
    for t in range(ZR // K):
        pltpu.sync_copy(msg_v, deg_sh.at[pl.ds(sid * ZR + t * K, K)])
    fill(1.0)
    plsc.subcore_barrier()

    pltpu.sync_copy(rc_hbm.at[cb], idx0)

    def body(i, carry):
        c0 = cb + 2 * i
        pltpu.async_copy(rc_hbm.at[c0 + 1], idx1, sem1)
        pltpu.sync_copy(msg_v, deg_sh.at[idx0.at[1]], add=True)

        @pl.when(i < NPAIR - 1)
        def _():
            pltpu.async_copy(rc_hbm.at[c0 + 2], idx0, sem0)

        pltpu.make_async_copy(rc_hbm.at[c0 + 1], idx1, sem1).wait()
        pltpu.sync_copy(msg_v, deg_sh.at[idx1.at[1]], add=True)

        @pl.when(i < NPAIR - 1)
        def _():
            pltpu.make_async_copy(rc_hbm.at[c0 + 2], idx0, sem0).wait()

        return carry

    lax.fori_loop(0, NPAIR, body, 0)
    plsc.subcore_barrier()

    pltpu.sync_copy(
        deg_sh.at[pl.ds(sid * ZR, ZR)], deg_hbm.at[cid].at[pl.ds(sid * ZR, ZR)]
    )


@functools.partial(
    pl.kernel,
    out_type=jax.ShapeDtypeStruct((NC, R16, D), jnp.float32),
    mesh=_mesh,
    scratch_types=[
        pltpu.VMEM((2, 2, K), jnp.int32),
        pltpu.VMEM((2, 2, K), jnp.int32),
        pltpu.VMEM((K, D), jnp.float32),
        pltpu.VMEM((K, D), jnp.float32),
        pltpu.VMEM_SHARED((R16, D), jnp.float32),
        pltpu.SemaphoreType.DMA,
        pltpu.SemaphoreType.DMA,
        pltpu.SemaphoreType.DMA,
        pltpu.SemaphoreType.DMA,
    ],
)
def _scatter_kernel(
    y_hbm, rcp_hbm, acc_hbm, pA, pB, msg0, msg1, acc_sh, sem0, sem1, semA, semB
):
    cid = lax.axis_index("c")
    sid = lax.axis_index("s")
    wid = sid * NC + cid
    pb = wid * NPAIR

    def fill_zeros(i, carry):
        for j in range(D // L):
            msg0[i, pl.ds(j * L, L)] = jnp.zeros((L,), jnp.float32)
        return carry

    lax.fori_loop(0, K, fill_zeros, 0)
    for t in range(ZR // K):
        pltpu.sync_copy(msg0, acc_sh.at[pl.ds(sid * ZR + t * K, K)])
    plsc.subcore_barrier()

    pltpu.sync_copy(rcp_hbm.at[pb], pA)
    pltpu.async_copy(y_hbm.at[pA.at[0, 0]], msg0, sem0)
    pltpu.async_copy(rcp_hbm.at[pb + 1], pB, semB)

    def quad(q, carry):
        pltpu.async_copy(y_hbm.at[pA.at[1, 0]], msg1, sem1)
        pltpu.make_async_copy(y_hbm.at[pA.at[0, 0]], msg0, sem0).wait()
        pltpu.sync_copy(msg0, acc_sh.at[pA.at[0, 1]], add=True)
        pltpu.make_async_copy(rcp_hbm.at[pb], pB, semB).wait()
        pltpu.async_copy(y_hbm.at[pB.at[0, 0]], msg0, sem0)
        pltpu.make_async_copy(y_hbm.at[pA.at[1, 0]], msg1, sem1).wait()
        pltpu.sync_copy(msg1, acc_sh.at[pA.at[1, 1]], add=True)

        @pl.when(q < NQUAD - 1)
        def _():
            pltpu.async_copy(rcp_hbm.at[pb + 2 * q + 2], pA, semA)

        pltpu.async_copy(y_hbm.at[pB.at[1, 0]], msg1, sem1)
        pltpu.make_async_copy(y_hbm.at[pB.at[0, 0]], msg0, sem0).wait()
        pltpu.sync_copy(msg0, acc_sh.at[pB.at[0, 1]], add=True)

        @pl.when(q < NQUAD - 1)
        def _():
            pltpu.make_async_copy(rcp_hbm.at[pb], pA, semA).wait()
            pltpu.async_copy(y_hbm.at[pA.at[0, 0]], msg0, sem0)

        pltpu.make_async_copy(y_hbm.at[pB.at[1, 0]], msg1, sem1).wait()
        pltpu.sync_copy(msg1, acc_sh.at[pB.at[1, 1]], add=True)

        @pl.when(q < NQUAD - 1)
        def _():
            pltpu.async_copy(rcp_hbm.at[pb + 2 * q + 3], pB, semB)

        return carry

    lax.fori_loop(0, NQUAD, quad, 0)
    plsc.subcore_barrier()
    pltpu.sync_copy(
        acc_sh.at[pl.ds(sid * ZR, ZR)], acc_hbm.at[cid].at[pl.ds(sid * ZR, ZR)]
    )


def _dis_from(deg_ref):
    s = deg_ref[0, :, 0:1] + deg_ref[1, :, 0:1] + 1.0
    return lax.rsqrt(s)


def _prep_body(x_ref, w_ref, deg_ref, y_ref):
    dis = _dis_from(deg_ref)
    y_ref[...] = (
        jnp.dot(x_ref[...], w_ref[...], preferred_element_type=jnp.float32) * dis
    )


_prep = pl.pallas_call(
    _prep_body,
    grid=(GRID,),
    in_specs=[
        pl.BlockSpec((BR, D), lambda m: (m, 0)),
        pl.BlockSpec((D, D), lambda m: (0, 0)),
        pl.BlockSpec((NC, BR, D), lambda m: (0, m, 0)),
    ],
    out_specs=pl.BlockSpec((BR, D), lambda m: (m, 0)),
    out_shape=jax.ShapeDtypeStruct((N, D), jnp.float32),
)


def _comb_mm_body(acc_ref, y_ref, deg_ref, b_ref, w_ref, out_ref):
    dis = _dis_from(deg_ref)
    h = jnp.maximum(
        (acc_ref[0] + acc_ref[1] + y_ref[...]) * dis + b_ref[...], 0.0
    )
    out_ref[...] = (
        jnp.dot(h, w_ref[...], preferred_element_type=jnp.float32) * dis
    )


_comb_mm = pl.pallas_call(
    _comb_mm_body,
    grid=(GRID,),
    in_specs=[
        pl.BlockSpec((NC, BR, D), lambda m: (0, m, 0)),
        pl.BlockSpec((BR, D), lambda m: (m, 0)),
        pl.BlockSpec((NC, BR, D), lambda m: (0, m, 0)),
        pl.BlockSpec((1, D), lambda m: (0, 0)),
        pl.BlockSpec((D, D), lambda m: (0, 0)),
    ],
    out_specs=pl.BlockSpec((BR, D), lambda m: (m, 0)),
    out_shape=jax.ShapeDtypeStruct((N, D), jnp.float32),
)


def _comb_body(acc_ref, y_ref, deg_ref, b_ref, out_ref):
    dis = _dis_from(deg_ref)
    out_ref[...] = jnp.maximum(
        (acc_ref[0] + acc_ref[1] + y_ref[...]) * dis + b_ref[...], 0.0
    )


_comb = pl.pallas_call(
    _comb_body,
    grid=(GRID,),
    in_specs=[
        pl.BlockSpec((NC, BR, D), lambda m: (0, m, 0)),
        pl.BlockSpec((BR, D), lambda m: (m, 0)),
        pl.BlockSpec((NC, BR, D), lambda m: (0, m, 0)),
        pl.BlockSpec((1, D), lambda m: (0, 0)),
    ],
    out_specs=pl.BlockSpec((BR, D), lambda m: (m, 0)),
    out_shape=jax.ShapeDtypeStruct((N, D), jnp.float32),
)


def kernel(x, edge_index, W1, b1, W2, b2):
    row = edge_index[0]
    col = edge_index[1]
    e = row.shape[0]
    pad = EP - e
    sprd = jnp.arange(pad, dtype=jnp.int32)
    row_p = jnp.concatenate([row, sprd % N])
    col_p = jnp.concatenate([col, N + sprd % (R16 - N)])
    rc = jnp.stack([row_p.reshape(TOTCH, K), col_p.reshape(TOTCH, K)], axis=1)
    rcp = rc.reshape(TOTCH // 2, 2, 2, K)

    degpair = _deg_kernel(rc)
    b1r = b1.reshape(1, D)
    b2r = b2.reshape(1, D)

    y1 = _prep(x, W1, degpair)
    accp1 = _scatter_kernel(y1, rcp)
    y2 = _comb_mm(accp1, y1, degpair, b1r, W2)
    accp2 = _scatter_kernel(y2, rcp)
    return _comb(accp2, y2, degpair, b2r)
